# baseline (speedup 1.0000x reference)
import jax
import jax.numpy as jnp
from jax import lax
from jax.experimental import pallas as pl
from jax.experimental.pallas import tpu as pltpu
from jax.experimental.pallas import tpu_sc as plsc

_N = 10000
_E = 320000
_P = 320000
_D = 128
_H = 128

_NC = 2
_NS = 16
_NW = _NC * _NS
_L = 16

_N_PAD = 10240
_DH = _D // 2
_EB = 64
_NBUF = 2
_PF = 1
_E_PAD = -(-_E // (_NW * _EB * _NBUF)) * (_NW * _EB * _NBUF)
_NBT = _E_PAD // _EB
_NB0 = 204
_NB1 = _NBT // _NS - _NB0
_NBPAD = _NB0 - _NB1
_RPT = _N_PAD // _NS
_PPT = _P // _NW
_BN = 1024

_mesh = plsc.VectorSubcoreMesh(core_axis_name="c", subcore_axis_name="s")


def _make_sc_aggregate(with_deg: bool):
    out_type = [jax.ShapeDtypeStruct((_NC, _N_PAD, _D), jnp.float32)]
    scratch = [
        pltpu.VMEM((_NB0, _EB), jnp.int32),
        pltpu.VMEM((_NB0, _EB), jnp.int32),
        pltpu.VMEM((_NBUF, _EB, _D), jnp.float32),
        pltpu.VMEM_SHARED((_N_PAD, _D), jnp.float32),
    ] + [pltpu.SemaphoreType.DMA] * (2 * _NBUF)
    if with_deg:
        out_type.append(jax.ShapeDtypeStruct((_NC, _N_PAD), jnp.float32))
        scratch = scratch + [
            pltpu.VMEM((_EB,), jnp.float32),
            pltpu.VMEM_SHARED((_N_PAD,), jnp.float32),
            pltpu.SemaphoreType.DMA,
        ]

    def body(table_hbm, src_hbm, dst_hbm, zerosh_hbm, zeros1_hbm, *refs):
        if with_deg:
            (agg_out, deg_out, src_v, dst_v, rows_v, agg_sh,
             *sems, ones_v, deg_sh, dsem) = refs
        else:
            agg_out, src_v, dst_v, rows_v, agg_sh, *sems = refs
        gsems, ssems = sems[:_NBUF], sems[_NBUF:]
        cid = lax.axis_index("c")
        sid = lax.axis_index("s")
        r0 = sid * _RPT
        base = jnp.where(cid == 0, sid * _NB0, _NS * _NB0 + sid * _NB1)
        nbm = jnp.where(cid == 0, _NB0, _NB1)
        pltpu.sync_copy(src_hbm.at[pl.ds(base, _NB0)], src_v)
        pltpu.sync_copy(dst_hbm.at[pl.ds(base, _NB0)], dst_v)
        pltpu.sync_copy(zerosh_hbm.at[pl.ds(r0, _RPT)],
                        agg_sh.at[pl.ds(r0, _RPT)])
        if with_deg:
            pltpu.sync_copy(zeros1_hbm.at[pl.ds(r0, _RPT)],
                            deg_sh.at[pl.ds(r0, _RPT)])
            for i in range(_EB // _L):
                ones_v[pl.ds(i * _L, _L)] = jnp.full((_L,), 1.0, jnp.float32)
        plsc.subcore_barrier()

        def gather(j, b):
            return pltpu.make_async_copy(table_hbm.at[src_v.at[b]],
                                         rows_v.at[j], gsems[j])

        def scatter(j, b):
            return pltpu.make_async_copy(rows_v.at[j],
                                         agg_sh.at[dst_v.at[b]], ssems[j])

        for j in range(_PF):
            gather(j, j).start()

        @pl.loop(0, nbm, step=_NBUF)
        def _(b0):
            for j in range(_NBUF):
                b = b0 + j
                pj = (j + _PF) % _NBUF
                pb = b + _PF

                @pl.when(pb < nbm)
                def _():
                    @pl.when(pb >= _NBUF)
                    def _():
                        scatter(pj, pb - _NBUF).wait()
                    gather(pj, pb).start()

                gather(j, b).wait()
                if with_deg:
                    @pl.when(b > 0)
                    def _():
                        pltpu.make_async_copy(
                            ones_v, deg_sh.at[dst_v.at[b]], dsem).wait()
                    pltpu.async_copy(ones_v, deg_sh.at[dst_v.at[b]], dsem,
                                     add=True)
                pltpu.async_copy(rows_v.at[j], agg_sh.at[dst_v.at[b]],
                                 ssems[j], add=True)

        for k in range(_NBUF):
            scatter(k, nbm - _NBUF + k).wait()
        if with_deg:
            pltpu.make_async_copy(
                ones_v, deg_sh.at[dst_v.at[nbm - 1]], dsem).wait()
        plsc.subcore_barrier()
        pltpu.sync_copy(agg_sh.at[pl.ds(r0, _RPT)],
                        agg_out.at[cid, pl.ds(r0, _RPT)])
        if with_deg:
            pltpu.sync_copy(deg_sh.at[pl.ds(r0, _RPT)],
                            deg_out.at[cid, pl.ds(r0, _RPT)])

    return pl.kernel(body, out_type=tuple(out_type), mesh=_mesh,
                     compiler_params=pltpu.CompilerParams(
                         use_tc_tiling_on_sc=False),
                     scratch_types=scratch)


_sc_aggregate_deg = _make_sc_aggregate(True)
_sc_aggregate = _make_sc_aggregate(False)


def _tc_layer_body(parts_ref, deg_ref, h_ref, wl_ref, bl_ref, wr_ref, out_ref):
    agg = parts_ref[0] + parts_ref[1]
    deg = deg_ref[0] + deg_ref[1]
    mean = agg * (1.0 / jnp.maximum(deg, 1.0))
    z = jnp.dot(mean, wl_ref[...], preferred_element_type=jnp.float32)
    z = z + bl_ref[...] + jnp.dot(h_ref[...], wr_ref[...],
                                  preferred_element_type=jnp.float32)
    out_ref[...] = jnp.maximum(z, 0.0)


def _tc_layer2_body(parts_ref, deg_ref, h_ref, wl_ref, bl_ref, wr_ref,
                    wst_ref, bst_ref, out_ref):
    agg = parts_ref[0] + parts_ref[1]
    h = h_ref[...]
    mean = agg * (1.0 / jnp.maximum(deg_ref[0] + deg_ref[1], 1.0))
    z = jnp.dot(mean, wl_ref[...], preferred_element_type=jnp.float32)
    z = z + bl_ref[...] + jnp.dot(h, wr_ref[...],
                                  preferred_element_type=jnp.float32)
    h2 = jnp.maximum(z, 0.0)
    out_ref[...] = jnp.dot(h2, wst_ref[...],
                           preferred_element_type=jnp.float32) + bst_ref[...]


_GRID = _N_PAD // _BN
_spec_parts = pl.BlockSpec((2, _BN, _D), lambda i: (0, i, 0))
_spec_rows = pl.BlockSpec((_BN, _D), lambda i: (i, 0))
_spec_deg = pl.BlockSpec((2, _BN, 1), lambda i: (0, i, 0))
_spec_w = pl.BlockSpec((_D, _H), lambda i: (0, 0))
_spec_b = pl.BlockSpec((1, _H), lambda i: (0, 0))

_tc_layer = pl.pallas_call(
    _tc_layer_body,
    out_shape=jax.ShapeDtypeStruct((_N_PAD, _D), jnp.float32),
    grid=(_GRID,),
    in_specs=[_spec_parts, _spec_deg, _spec_rows, _spec_w, _spec_b,
              _spec_w],
    out_specs=_spec_rows,
    compiler_params=pltpu.CompilerParams(
        dimension_semantics=("parallel",)),
)

_tc_layer2 = pl.pallas_call(
    _tc_layer2_body,
    out_shape=jax.ShapeDtypeStruct((_N_PAD, 2), jnp.float32),
    grid=(_GRID,),
    in_specs=[_spec_parts, _spec_deg, _spec_rows, _spec_w, _spec_b,
              _spec_w,
              pl.BlockSpec((_D, 2), lambda i: (0, 0)),
              pl.BlockSpec((1, 2), lambda i: (0, 0))],
    out_specs=pl.BlockSpec((_BN, 2), lambda i: (i, 0)),
    compiler_params=pltpu.CompilerParams(
        dimension_semantics=("parallel",)),
)


def _sc_pairs_body(s_hbm, t_hbm, p0_hbm, p1_hbm, out_hbm,
                   s_v, t_v, p0_v, p1_v, out_v):
    cid = lax.axis_index("c")
    sid = lax.axis_index("s")
    wid = sid * _NC + cid
    base = wid * _PPT
    pltpu.sync_copy(s_hbm, s_v)
    pltpu.sync_copy(t_hbm, t_v)
    pltpu.sync_copy(p0_hbm.at[pl.ds(base, _PPT)], p0_v)
    pltpu.sync_copy(p1_hbm.at[pl.ds(base, _PPT)], p1_v)

    def step(i, carry):
        o = i * _L
        i0 = p0_v[pl.ds(o, _L)]
        i1 = p1_v[pl.ds(o, _L)]
        out_v[pl.ds(o, _L)] = (plsc.load_gather(s_v, [i0]) +
                               plsc.load_gather(t_v, [i1]))
        return carry

    lax.fori_loop(0, _PPT // _L, step, 0)
    pltpu.sync_copy(out_v, out_hbm.at[pl.ds(base, _PPT)])


_sc_pairs = pl.kernel(
    _sc_pairs_body,
    out_type=jax.ShapeDtypeStruct((_P,), jnp.float32),
    mesh=_mesh,
    compiler_params=pltpu.CompilerParams(needs_layout_passes=False),
    scratch_types=[
        pltpu.VMEM((_N_PAD,), jnp.float32),
        pltpu.VMEM((_N_PAD,), jnp.float32),
        pltpu.VMEM((_PPT,), jnp.int32),
        pltpu.VMEM((_PPT,), jnp.int32),
        pltpu.VMEM((_PPT,), jnp.float32),
    ],
)


def kernel(x, edge_index, edge_pairs, W_l1, b_l1, W_r1, W_l2, b_l2, W_r2,
           W_head, b_head):
    epad = (_NBT + _NBPAD) * _EB - _E
    src = jnp.concatenate([edge_index[0], jnp.zeros((epad,), jnp.int32)])
    dst = jnp.concatenate([edge_index[1], jnp.full((epad,), _N, jnp.int32)])
    src = src.reshape(_NBT + _NBPAD, _EB)
    dst = dst.reshape(_NBT + _NBPAD, _EB)
    x_p = jnp.pad(x, ((0, _N_PAD - _N), (0, 0)))
    zerosh = jnp.zeros((_N_PAD, _D), jnp.float32)
    zeros1 = jnp.zeros((_N_PAD,), jnp.float32)

    agg1, deg = _sc_aggregate_deg(x_p, src, dst, zerosh, zeros1)
    deg2 = deg.reshape(_NC, _N_PAD, 1)
    h1 = _tc_layer(agg1, deg2, x_p, W_l1.T, b_l1[None, :], W_r1.T)

    (agg2,) = _sc_aggregate(h1, src, dst, zerosh, zeros1)
    wst = jnp.stack([W_head[0, :_H], W_head[0, _H:]], axis=1)
    bst = jnp.stack([jnp.zeros((), jnp.float32), b_head[0]])[None, :]
    st = _tc_layer2(agg2, deg2, h1, W_l2.T, b_l2[None, :], W_r2.T, wst, bst)

    s = st[:, 0]
    t = st[:, 1]
    p0 = edge_pairs[:, 0]
    p1 = edge_pairs[:, 1]
    return _sc_pairs(s, t, p0, p1)


# speedup vs baseline: 21.1197x; 21.1197x over previous
import jax
import jax.numpy as jnp
from jax import lax
from jax.experimental import pallas as pl
from jax.experimental.pallas import tpu as pltpu
from jax.experimental.pallas import tpu_sc as plsc

_N = 10000
_E = 320000
_P = 320000
_D = 128
_H = 128

_NC = 2
_NS = 16
_NW = _NC * _NS
_L = 16

_N_PAD = 10240
_DH = _D // 2
_EB = 64
_NBUF = 2
_PF = 1
_E_PAD = -(-_E // (_NW * _EB * _NBUF)) * (_NW * _EB * _NBUF)
_NBT = _E_PAD // _EB
_NB0 = 204
_NB1 = _NBT // _NS - _NB0
_NBPAD = _NB0 - _NB1
_RPT = _N_PAD // _NS
_PPT = _P // _NW
_BN = 1024

_mesh = plsc.VectorSubcoreMesh(core_axis_name="c", subcore_axis_name="s")


def _make_sc_aggregate(with_deg: bool):
    out_type = [jax.ShapeDtypeStruct((_NC, _N_PAD, _D), jnp.float32)]
    scratch = [
        pltpu.VMEM((_NB0, _EB), jnp.int32),
        pltpu.VMEM((_NB0, _EB), jnp.int32),
        pltpu.VMEM((_NBUF, _EB, _D), jnp.float32),
        pltpu.VMEM_SHARED((_N_PAD, _D), jnp.float32),
    ] + [pltpu.SemaphoreType.DMA] * (2 * _NBUF)
    if with_deg:
        out_type.append(jax.ShapeDtypeStruct((_NC, _N_PAD), jnp.float32))
        scratch = scratch + [
            pltpu.VMEM((_EB,), jnp.float32),
            pltpu.VMEM_SHARED((_N_PAD,), jnp.float32),
            pltpu.SemaphoreType.DMA,
        ]

    def body(table_hbm, src_hbm, dst_hbm, zerosh_hbm, zeros1_hbm, *refs):
        if with_deg:
            (agg_out, deg_out, src_v, dst_v, rows_v, agg_sh,
             *sems, ones_v, deg_sh, dsem) = refs
        else:
            agg_out, src_v, dst_v, rows_v, agg_sh, *sems = refs
        gsems, ssems = sems[:_NBUF], sems[_NBUF:]
        cid = lax.axis_index("c")
        sid = lax.axis_index("s")
        r0 = sid * _RPT
        base = jnp.where(cid == 0, sid * _NB0, _NS * _NB0 + sid * _NB1)
        pltpu.sync_copy(src_hbm.at[pl.ds(base, _NB0)], src_v)
        pltpu.sync_copy(dst_hbm.at[pl.ds(base, _NB0)], dst_v)
        pltpu.sync_copy(zerosh_hbm.at[pl.ds(r0, _RPT)],
                        agg_sh.at[pl.ds(r0, _RPT)])
        if with_deg:
            pltpu.sync_copy(zeros1_hbm.at[pl.ds(r0, _RPT)],
                            deg_sh.at[pl.ds(r0, _RPT)])
            for i in range(_EB // _L):
                ones_v[pl.ds(i * _L, _L)] = jnp.full((_L,), 1.0, jnp.float32)
        plsc.subcore_barrier()

        def gather(j, b):
            return pltpu.make_async_copy(table_hbm.at[src_v.at[b]],
                                         rows_v.at[j], gsems[j])

        def scatter(j, b):
            return pltpu.make_async_copy(rows_v.at[j],
                                         agg_sh.at[dst_v.at[b]], ssems[j])

        def run(nb):
            for j in range(_PF):
                gather(j, j).start()

            @pl.loop(0, nb, step=_NBUF)
            def _(b0):
                for j in range(_NBUF):
                    b = b0 + j
                    pj = (j + _PF) % _NBUF
                    pb = b + _PF

                    @pl.when(pb < nb)
                    def _():
                        @pl.when(pb >= _NBUF)
                        def _():
                            scatter(pj, pb - _NBUF).wait()
                        gather(pj, pb).start()

                    gather(j, b).wait()
                    if with_deg:
                        @pl.when(b > 0)
                        def _():
                            pltpu.make_async_copy(
                                ones_v, deg_sh.at[dst_v.at[b]], dsem).wait()
                        pltpu.async_copy(ones_v, deg_sh.at[dst_v.at[b]],
                                         dsem, add=True)
                    pltpu.async_copy(rows_v.at[j], agg_sh.at[dst_v.at[b]],
                                     ssems[j], add=True)

            for k in range(_NBUF):
                scatter(k, nb - _NBUF + k).wait()
            if with_deg:
                pltpu.make_async_copy(
                    ones_v, deg_sh.at[dst_v.at[nb - 1]], dsem).wait()

        @pl.when(cid == 0)
        def _():
            run(_NB0)

        @pl.when(cid == 1)
        def _():
            run(_NB1)

        plsc.subcore_barrier()
        pltpu.sync_copy(agg_sh.at[pl.ds(r0, _RPT)],
                        agg_out.at[cid, pl.ds(r0, _RPT)])
        if with_deg:
            pltpu.sync_copy(deg_sh.at[pl.ds(r0, _RPT)],
                            deg_out.at[cid, pl.ds(r0, _RPT)])

    return pl.kernel(body, out_type=tuple(out_type), mesh=_mesh,
                     compiler_params=pltpu.CompilerParams(
                         use_tc_tiling_on_sc=False),
                     scratch_types=scratch)


_sc_aggregate_deg = _make_sc_aggregate(True)
_sc_aggregate = _make_sc_aggregate(False)


def _tc_layer_body(parts_ref, deg_ref, h_ref, wl_ref, bl_ref, wr_ref, out_ref):
    agg = parts_ref[0] + parts_ref[1]
    deg = deg_ref[0] + deg_ref[1]
    mean = agg * (1.0 / jnp.maximum(deg, 1.0))
    z = jnp.dot(mean, wl_ref[...], preferred_element_type=jnp.float32)
    z = z + bl_ref[...] + jnp.dot(h_ref[...], wr_ref[...],
                                  preferred_element_type=jnp.float32)
    out_ref[...] = jnp.maximum(z, 0.0)


def _tc_layer2_body(parts_ref, deg_ref, h_ref, wl_ref, bl_ref, wr_ref,
                    wst_ref, bst_ref, out_ref):
    agg = parts_ref[0] + parts_ref[1]
    h = h_ref[...]
    mean = agg * (1.0 / jnp.maximum(deg_ref[0] + deg_ref[1], 1.0))
    z = jnp.dot(mean, wl_ref[...], preferred_element_type=jnp.float32)
    z = z + bl_ref[...] + jnp.dot(h, wr_ref[...],
                                  preferred_element_type=jnp.float32)
    h2 = jnp.maximum(z, 0.0)
    out_ref[...] = jnp.dot(h2, wst_ref[...],
                           preferred_element_type=jnp.float32) + bst_ref[...]


_GRID = _N_PAD // _BN
_spec_parts = pl.BlockSpec((2, _BN, _D), lambda i: (0, i, 0))
_spec_rows = pl.BlockSpec((_BN, _D), lambda i: (i, 0))
_spec_deg = pl.BlockSpec((2, _BN, 1), lambda i: (0, i, 0))
_spec_w = pl.BlockSpec((_D, _H), lambda i: (0, 0))
_spec_b = pl.BlockSpec((1, _H), lambda i: (0, 0))

_tc_layer = pl.pallas_call(
    _tc_layer_body,
    out_shape=jax.ShapeDtypeStruct((_N_PAD, _D), jnp.float32),
    grid=(_GRID,),
    in_specs=[_spec_parts, _spec_deg, _spec_rows, _spec_w, _spec_b,
              _spec_w],
    out_specs=_spec_rows,
    compiler_params=pltpu.CompilerParams(
        dimension_semantics=("parallel",)),
)

_tc_layer2 = pl.pallas_call(
    _tc_layer2_body,
    out_shape=jax.ShapeDtypeStruct((_N_PAD, 2), jnp.float32),
    grid=(_GRID,),
    in_specs=[_spec_parts, _spec_deg, _spec_rows, _spec_w, _spec_b,
              _spec_w,
              pl.BlockSpec((_D, 2), lambda i: (0, 0)),
              pl.BlockSpec((1, 2), lambda i: (0, 0))],
    out_specs=pl.BlockSpec((_BN, 2), lambda i: (i, 0)),
    compiler_params=pltpu.CompilerParams(
        dimension_semantics=("parallel",)),
)


def _sc_pairs_body(s_hbm, t_hbm, p0_hbm, p1_hbm, out_hbm,
                   s_v, t_v, p0_v, p1_v, out_v):
    cid = lax.axis_index("c")
    sid = lax.axis_index("s")
    wid = sid * _NC + cid
    base = wid * _PPT
    pltpu.sync_copy(s_hbm, s_v)
    pltpu.sync_copy(t_hbm, t_v)
    pltpu.sync_copy(p0_hbm.at[pl.ds(base, _PPT)], p0_v)
    pltpu.sync_copy(p1_hbm.at[pl.ds(base, _PPT)], p1_v)

    def step(i, carry):
        o = i * _L
        i0 = p0_v[pl.ds(o, _L)]
        i1 = p1_v[pl.ds(o, _L)]
        out_v[pl.ds(o, _L)] = (plsc.load_gather(s_v, [i0]) +
                               plsc.load_gather(t_v, [i1]))
        return carry

    lax.fori_loop(0, _PPT // _L, step, 0)
    pltpu.sync_copy(out_v, out_hbm.at[pl.ds(base, _PPT)])


_sc_pairs = pl.kernel(
    _sc_pairs_body,
    out_type=jax.ShapeDtypeStruct((_P,), jnp.float32),
    mesh=_mesh,
    compiler_params=pltpu.CompilerParams(needs_layout_passes=False),
    scratch_types=[
        pltpu.VMEM((_N_PAD,), jnp.float32),
        pltpu.VMEM((_N_PAD,), jnp.float32),
        pltpu.VMEM((_PPT,), jnp.int32),
        pltpu.VMEM((_PPT,), jnp.int32),
        pltpu.VMEM((_PPT,), jnp.float32),
    ],
)


def kernel(x, edge_index, edge_pairs, W_l1, b_l1, W_r1, W_l2, b_l2, W_r2,
           W_head, b_head):
    epad = (_NBT + _NBPAD) * _EB - _E
    src = jnp.concatenate([edge_index[0], jnp.zeros((epad,), jnp.int32)])
    dst = jnp.concatenate([edge_index[1], jnp.full((epad,), _N, jnp.int32)])
    src = src.reshape(_NBT + _NBPAD, _EB)
    dst = dst.reshape(_NBT + _NBPAD, _EB)
    x_p = jnp.pad(x, ((0, _N_PAD - _N), (0, 0)))
    zerosh = jnp.zeros((_N_PAD, _D), jnp.float32)
    zeros1 = jnp.zeros((_N_PAD,), jnp.float32)

    agg1, deg = _sc_aggregate_deg(x_p, src, dst, zerosh, zeros1)
    deg2 = deg.reshape(_NC, _N_PAD, 1)
    h1 = _tc_layer(agg1, deg2, x_p, W_l1.T, b_l1[None, :], W_r1.T)

    (agg2,) = _sc_aggregate(h1, src, dst, zerosh, zeros1)
    wst = jnp.stack([W_head[0, :_H], W_head[0, _H:]], axis=1)
    bst = jnp.stack([jnp.zeros((), jnp.float32), b_head[0]])[None, :]
    st = _tc_layer2(agg2, deg2, h1, W_l2.T, b_l2[None, :], W_r2.T, wst, bst)

    s = st[:, 0]
    t = st[:, 1]
    p0 = edge_pairs[:, 0]
    p1 = edge_pairs[:, 1]
    return _sc_pairs(s, t, p0, p1)


# speedup vs baseline: 21.7592x; 1.0303x over previous
import jax
import jax.numpy as jnp
from jax import lax
from jax.experimental import pallas as pl
from jax.experimental.pallas import tpu as pltpu
from jax.experimental.pallas import tpu_sc as plsc

_N = 10000
_E = 320000
_P = 320000
_D = 128
_H = 128

_NC = 2
_NS = 16
_NW = _NC * _NS
_L = 16

_N_PAD = 10240
_DH = _D // 2
_EB = 64
_NBUF = 2
_PF = 1
_E_PAD = -(-_E // (_NW * _EB * _NBUF)) * (_NW * _EB * _NBUF)
_NBT = _E_PAD // _EB
_NB0 = 232
_NB1 = _NBT // _NS - _NB0
_NBPAD = _NB0 - _NB1
_RPT = _N_PAD // _NS
_PPT = _P // _NW
_BN = 1024

_mesh = plsc.VectorSubcoreMesh(core_axis_name="c", subcore_axis_name="s")


def _make_sc_aggregate(with_deg: bool):
    out_type = [jax.ShapeDtypeStruct((_NC, _N_PAD, _D), jnp.float32)]
    scratch = [
        pltpu.VMEM((_NB0, _EB), jnp.int32),
        pltpu.VMEM((_NB0, _EB), jnp.int32),
        pltpu.VMEM((_NBUF, _EB, _D), jnp.float32),
        pltpu.VMEM_SHARED((_N_PAD, _D), jnp.float32),
    ] + [pltpu.SemaphoreType.DMA] * (2 * _NBUF)
    if with_deg:
        out_type.append(jax.ShapeDtypeStruct((_NC, _N_PAD), jnp.float32))
        scratch = scratch + [
            pltpu.VMEM((_EB,), jnp.float32),
            pltpu.VMEM_SHARED((_N_PAD,), jnp.float32),
            pltpu.SemaphoreType.DMA,
        ]

    def body(table_hbm, src_hbm, dst_hbm, zerosh_hbm, zeros1_hbm, *refs):
        if with_deg:
            (agg_out, deg_out, src_v, dst_v, rows_v, agg_sh,
             *sems, ones_v, deg_sh, dsem) = refs
        else:
            agg_out, src_v, dst_v, rows_v, agg_sh, *sems = refs
        gsems, ssems = sems[:_NBUF], sems[_NBUF:]
        cid = lax.axis_index("c")
        sid = lax.axis_index("s")
        r0 = sid * _RPT
        base = jnp.where(cid == 0, sid * _NB0, _NS * _NB0 + sid * _NB1)
        pltpu.sync_copy(src_hbm.at[pl.ds(base, _NB0)], src_v)
        pltpu.sync_copy(dst_hbm.at[pl.ds(base, _NB0)], dst_v)
        for e in range(_EB):
            for c in range(_D // _L):
                rows_v[0, e, pl.ds(c * _L, _L)] = jnp.zeros((_L,), jnp.float32)
        for k in range(_RPT // _EB):
            pltpu.sync_copy(rows_v.at[0],
                            agg_sh.at[pl.ds(r0 + k * _EB, _EB)])
        if with_deg:
            pltpu.sync_copy(zeros1_hbm.at[pl.ds(r0, _RPT)],
                            deg_sh.at[pl.ds(r0, _RPT)])
            for i in range(_EB // _L):
                ones_v[pl.ds(i * _L, _L)] = jnp.full((_L,), 1.0, jnp.float32)
        plsc.subcore_barrier()

        def gather(j, b):
            return pltpu.make_async_copy(table_hbm.at[src_v.at[b]],
                                         rows_v.at[j], gsems[j])

        def scatter(j, b):
            return pltpu.make_async_copy(rows_v.at[j],
                                         agg_sh.at[dst_v.at[b]], ssems[j])

        def run(nb):
            for j in range(_PF):
                gather(j, j).start()

            @pl.loop(0, nb, step=_NBUF)
            def _(b0):
                for j in range(_NBUF):
                    b = b0 + j
                    pj = (j + _PF) % _NBUF
                    pb = b + _PF

                    @pl.when(pb < nb)
                    def _():
                        @pl.when(pb >= _NBUF)
                        def _():
                            scatter(pj, pb - _NBUF).wait()
                        gather(pj, pb).start()

                    gather(j, b).wait()
                    if with_deg:
                        @pl.when(b > 0)
                        def _():
                            pltpu.make_async_copy(
                                ones_v, deg_sh.at[dst_v.at[b]], dsem).wait()
                        pltpu.async_copy(ones_v, deg_sh.at[dst_v.at[b]],
                                         dsem, add=True)
                    pltpu.async_copy(rows_v.at[j], agg_sh.at[dst_v.at[b]],
                                     ssems[j], add=True)

            for k in range(_NBUF):
                scatter(k, nb - _NBUF + k).wait()
            if with_deg:
                pltpu.make_async_copy(
                    ones_v, deg_sh.at[dst_v.at[nb - 1]], dsem).wait()

        @pl.when(cid == 0)
        def _():
            run(_NB0)

        @pl.when(cid == 1)
        def _():
            run(_NB1)

        plsc.subcore_barrier()
        pltpu.sync_copy(agg_sh.at[pl.ds(r0, _RPT)],
                        agg_out.at[cid, pl.ds(r0, _RPT)])
        if with_deg:
            pltpu.sync_copy(deg_sh.at[pl.ds(r0, _RPT)],
                            deg_out.at[cid, pl.ds(r0, _RPT)])

    return pl.kernel(body, out_type=tuple(out_type), mesh=_mesh,
                     compiler_params=pltpu.CompilerParams(
                         use_tc_tiling_on_sc=False),
                     scratch_types=scratch)


_sc_aggregate_deg = _make_sc_aggregate(True)
_sc_aggregate = _make_sc_aggregate(False)


def _tc_layer_body(parts_ref, deg_ref, h_ref, wl_ref, bl_ref, wr_ref, out_ref):
    agg = parts_ref[0] + parts_ref[1]
    deg = deg_ref[0] + deg_ref[1]
    mean = agg * (1.0 / jnp.maximum(deg, 1.0))
    z = jnp.dot(mean, wl_ref[...], preferred_element_type=jnp.float32)
    z = z + bl_ref[...] + jnp.dot(h_ref[...], wr_ref[...],
                                  preferred_element_type=jnp.float32)
    out_ref[...] = jnp.maximum(z, 0.0)


def _tc_layer2_body(parts_ref, deg_ref, h_ref, wl_ref, bl_ref, wr_ref,
                    wst_ref, bst_ref, out_ref):
    agg = parts_ref[0] + parts_ref[1]
    h = h_ref[...]
    mean = agg * (1.0 / jnp.maximum(deg_ref[0] + deg_ref[1], 1.0))
    z = jnp.dot(mean, wl_ref[...], preferred_element_type=jnp.float32)
    z = z + bl_ref[...] + jnp.dot(h, wr_ref[...],
                                  preferred_element_type=jnp.float32)
    h2 = jnp.maximum(z, 0.0)
    out_ref[...] = jnp.dot(h2, wst_ref[...],
                           preferred_element_type=jnp.float32) + bst_ref[...]


_GRID = _N_PAD // _BN
_spec_parts = pl.BlockSpec((2, _BN, _D), lambda i: (0, i, 0))
_spec_rows = pl.BlockSpec((_BN, _D), lambda i: (i, 0))
_spec_deg = pl.BlockSpec((2, _BN, 1), lambda i: (0, i, 0))
_spec_w = pl.BlockSpec((_D, _H), lambda i: (0, 0))
_spec_b = pl.BlockSpec((1, _H), lambda i: (0, 0))

_tc_layer = pl.pallas_call(
    _tc_layer_body,
    out_shape=jax.ShapeDtypeStruct((_N_PAD, _D), jnp.float32),
    grid=(_GRID,),
    in_specs=[_spec_parts, _spec_deg, _spec_rows, _spec_w, _spec_b,
              _spec_w],
    out_specs=_spec_rows,
    compiler_params=pltpu.CompilerParams(
        dimension_semantics=("parallel",)),
)

_tc_layer2 = pl.pallas_call(
    _tc_layer2_body,
    out_shape=jax.ShapeDtypeStruct((_N_PAD, 2), jnp.float32),
    grid=(_GRID,),
    in_specs=[_spec_parts, _spec_deg, _spec_rows, _spec_w, _spec_b,
              _spec_w,
              pl.BlockSpec((_D, 2), lambda i: (0, 0)),
              pl.BlockSpec((1, 2), lambda i: (0, 0))],
    out_specs=pl.BlockSpec((_BN, 2), lambda i: (i, 0)),
    compiler_params=pltpu.CompilerParams(
        dimension_semantics=("parallel",)),
)


def _sc_pairs_body(s_hbm, t_hbm, p0_hbm, p1_hbm, out_hbm,
                   s_v, t_v, p0_v, p1_v, out_v):
    cid = lax.axis_index("c")
    sid = lax.axis_index("s")
    wid = sid * _NC + cid
    base = wid * _PPT
    pltpu.sync_copy(s_hbm, s_v)
    pltpu.sync_copy(t_hbm, t_v)
    pltpu.sync_copy(p0_hbm.at[pl.ds(base, _PPT)], p0_v)
    pltpu.sync_copy(p1_hbm.at[pl.ds(base, _PPT)], p1_v)

    def step(i, carry):
        o = i * _L
        i0 = p0_v[pl.ds(o, _L)]
        i1 = p1_v[pl.ds(o, _L)]
        out_v[pl.ds(o, _L)] = (plsc.load_gather(s_v, [i0]) +
                               plsc.load_gather(t_v, [i1]))
        return carry

    lax.fori_loop(0, _PPT // _L, step, 0)
    pltpu.sync_copy(out_v, out_hbm.at[pl.ds(base, _PPT)])


_sc_pairs = pl.kernel(
    _sc_pairs_body,
    out_type=jax.ShapeDtypeStruct((_P,), jnp.float32),
    mesh=_mesh,
    compiler_params=pltpu.CompilerParams(needs_layout_passes=False),
    scratch_types=[
        pltpu.VMEM((_N_PAD,), jnp.float32),
        pltpu.VMEM((_N_PAD,), jnp.float32),
        pltpu.VMEM((_PPT,), jnp.int32),
        pltpu.VMEM((_PPT,), jnp.int32),
        pltpu.VMEM((_PPT,), jnp.float32),
    ],
)


def kernel(x, edge_index, edge_pairs, W_l1, b_l1, W_r1, W_l2, b_l2, W_r2,
           W_head, b_head):
    epad = (_NBT + _NBPAD) * _EB - _E
    src = jnp.concatenate([edge_index[0], jnp.zeros((epad,), jnp.int32)])
    dst = jnp.concatenate([edge_index[1], jnp.full((epad,), _N, jnp.int32)])
    src = src.reshape(_NBT + _NBPAD, _EB)
    dst = dst.reshape(_NBT + _NBPAD, _EB)
    x_p = jnp.pad(x, ((0, _N_PAD - _N), (0, 0)))
    zerosh = jnp.zeros((_N_PAD, _D), jnp.float32)
    zeros1 = jnp.zeros((_N_PAD,), jnp.float32)

    agg1, deg = _sc_aggregate_deg(x_p, src, dst, zerosh, zeros1)
    deg2 = deg.reshape(_NC, _N_PAD, 1)
    h1 = _tc_layer(agg1, deg2, x_p, W_l1.T, b_l1[None, :], W_r1.T)

    (agg2,) = _sc_aggregate(h1, src, dst, zerosh, zeros1)
    wst = jnp.stack([W_head[0, :_H], W_head[0, _H:]], axis=1)
    bst = jnp.stack([jnp.zeros((), jnp.float32), b_head[0]])[None, :]
    st = _tc_layer2(agg2, deg2, h1, W_l2.T, b_l2[None, :], W_r2.T, wst, bst)

    s = st[:, 0]
    t = st[:, 1]
    p0 = edge_pairs[:, 0]
    p1 = edge_pairs[:, 1]
    return _sc_pairs(s, t, p0, p1)
